# baseline (speedup 1.0000x reference)
import functools

import jax
import jax.numpy as jnp
from jax import lax
from jax.experimental import pallas as pl
from jax.experimental.pallas import tpu as pltpu
from jax.experimental.pallas import tpu_sc as plsc

N = 10000
E = 320000
F = 128
NCLASS = 40

NC = 2
NS = 16
NW = NC * NS
CH = 128
NPAD = 10240
RPT = NPAD // NS
NCHUNKP = 2560
CPW = NCHUNKP // NW
CPT = NCHUNKP // NS
NB = 40

_MESH = plsc.VectorSubcoreMesh(core_axis_name="c", subcore_axis_name="s")


def _fill_vmem(ref, nrows, width, value):
    vv = jnp.full((16,), value, jnp.float32)
    def body(r, _):
        for j in range(width // 16):
            ref[r, pl.ds(j * 16, 16)] = vv
        return ()
    lax.fori_loop(0, nrows, body, ())


@functools.partial(
    pl.kernel,
    out_type=jax.ShapeDtypeStruct((2, NPAD), jnp.float32),
    mesh=_MESH,
    scratch_types=[
        pltpu.VMEM_SHARED((NPAD,), jnp.float32),
        pltpu.VMEM((NB, CH), jnp.int32),
        pltpu.VMEM((CH,), jnp.float32),
        pltpu.VMEM((RPT,), jnp.float32),
    ],
)
def _sc_degrees(ei2, out, acc, idxb, ones1, zbuf):
    cid = lax.axis_index("c")
    tid = lax.axis_index("s")

    zv = jnp.zeros((16,), jnp.float32)
    def z(r, _):
        zbuf[pl.ds(r * 16, 16)] = zv
        return ()
    lax.fori_loop(0, RPT // 16, z, ())
    pltpu.sync_copy(zbuf, acc.at[pl.ds(tid * RPT, RPT)])

    ov = jnp.ones((16,), jnp.float32)
    def o(r, _):
        ones1[pl.ds(r * 16, 16)] = ov
        return ()
    lax.fori_loop(0, CH // 16, o, ())

    plsc.subcore_barrier()

    base = cid * NCHUNKP + tid * CPT
    def batch(b, _):
        pltpu.sync_copy(ei2.at[pl.ds(base + b * NB, NB)], idxb)
        def step(j, _):
            pltpu.sync_copy(ones1, acc.at[idxb.at[j]], add=True)
            return ()
        lax.fori_loop(0, NB, step, ())
        return ()
    lax.fori_loop(0, CPT // NB, batch, ())

    plsc.subcore_barrier()
    pltpu.sync_copy(acc.at[pl.ds(tid * RPT, RPT)],
                    out.at[cid, pl.ds(tid * RPT, RPT)])


@functools.partial(
    pl.kernel,
    out_type=jax.ShapeDtypeStruct((NC, NPAD, F), jnp.float32),
    mesh=_MESH,
    scratch_types=[
        pltpu.VMEM_SHARED((NPAD, F), jnp.float32),
        pltpu.VMEM((NB, CH), jnp.int32),
        pltpu.VMEM((NB, CH), jnp.int32),
        pltpu.VMEM((CH, F), jnp.float32),
        pltpu.VMEM((CH, F), jnp.float32),
        pltpu.SemaphoreType.DMA,
        pltpu.SemaphoreType.DMA,
    ],
)
def _sc_aggregate(xs, ei2, out, acc, sidx, didx, rows0, rows1, sem0, sem1):
    cid = lax.axis_index("c")
    tid = lax.axis_index("s")
    wid = tid * NC + cid

    _fill_vmem(rows0, CH, F, 0.0)
    for k in range(RPT // CH):
        pltpu.sync_copy(rows0, acc.at[pl.ds(tid * RPT + k * CH, CH)])
    plsc.subcore_barrier()

    base = wid * CPW
    def half(h, _):
        hb = base + h * NB
        pltpu.sync_copy(ei2.at[pl.ds(hb, NB)], sidx)
        pltpu.sync_copy(ei2.at[pl.ds(NCHUNKP + hb, NB)], didx)
        pltpu.async_copy(xs.at[sidx.at[0]], rows0, sem0)
        def body(j2, _):
            c0 = 2 * j2
            pltpu.async_copy(xs.at[sidx.at[c0 + 1]], rows1, sem1)
            pltpu.make_async_copy(xs.at[sidx.at[c0]], rows0, sem0).wait()
            pltpu.sync_copy(rows0, acc.at[didx.at[c0]], add=True)
            @pl.when(j2 < NB // 2 - 1)
            def _():
                pltpu.async_copy(xs.at[sidx.at[c0 + 2]], rows0, sem0)
            pltpu.make_async_copy(xs.at[sidx.at[c0 + 1]], rows1, sem1).wait()
            pltpu.sync_copy(rows1, acc.at[didx.at[c0 + 1]], add=True)
            return ()
        lax.fori_loop(0, NB // 2, body, ())
        return ()
    lax.fori_loop(0, CPW // NB, half, ())

    plsc.subcore_barrier()
    pltpu.sync_copy(acc.at[pl.ds(tid * RPT, RPT)],
                    out.at[cid, pl.ds(tid * RPT, RPT)])


def _norm_from(deg_ref, which):
    d = deg_ref[which, :N]
    return jnp.where(d > 0.0, lax.rsqrt(jnp.maximum(d, 1.0)), 0.0)


def _tc_prescale_body(x_ref, deg_ref, o_ref):
    o_ref[:N] = x_ref[...] * _norm_from(deg_ref, 0)
    o_ref[N:] = jnp.zeros((NPAD - N, F), jnp.float32)


def _tc_layer1_body(p_ref, deg_ref, w_ref, b_ref, o_ref):
    y = (p_ref[0, :N] + p_ref[1, :N]) * _norm_from(deg_ref, 1)
    h = jnp.dot(y, w_ref[...], preferred_element_type=jnp.float32) + b_ref[...]
    h = jnp.maximum(h, 0.0)
    o_ref[:N] = h * _norm_from(deg_ref, 0)
    o_ref[N:] = jnp.zeros((NPAD - N, F), jnp.float32)


def _tc_layer2_body(p_ref, deg_ref, w_ref, b_ref, wfc_ref, bfc_ref, o_ref):
    y = (p_ref[0, :N] + p_ref[1, :N]) * _norm_from(deg_ref, 1)
    h = jnp.dot(y, w_ref[...], preferred_element_type=jnp.float32) + b_ref[...]
    o_ref[...] = (jnp.dot(h, wfc_ref[...], preferred_element_type=jnp.float32)
                  + bfc_ref[...])


def kernel(x, edge_index, W1, b1, W2, b2, Wfc, bfc):
    padv = jnp.full((NCHUNKP * CH - E,), NPAD - 1, jnp.int32)
    ei2 = jnp.concatenate(
        [edge_index[0], padv, edge_index[1], padv]).reshape(2 * NCHUNKP, CH)

    degp = _sc_degrees(ei2)
    degc = degp.reshape(2, NPAD, 1)

    xs1 = pl.pallas_call(
        _tc_prescale_body,
        out_shape=jax.ShapeDtypeStruct((NPAD, F), jnp.float32),
    )(x, degc)

    P1 = _sc_aggregate(xs1, ei2)

    xs2 = pl.pallas_call(
        _tc_layer1_body,
        out_shape=jax.ShapeDtypeStruct((NPAD, F), jnp.float32),
    )(P1, degc, W1, b1.reshape(1, F))

    P2 = _sc_aggregate(xs2, ei2)

    out = pl.pallas_call(
        _tc_layer2_body,
        out_shape=jax.ShapeDtypeStruct((N, NCLASS), jnp.float32),
    )(P2, degc, W2, b2.reshape(1, F), Wfc, bfc.reshape(1, NCLASS))

    return out


# speedup vs baseline: 3.5648x; 3.5648x over previous
import functools

import jax
import jax.numpy as jnp
from jax import lax
from jax.experimental import pallas as pl
from jax.experimental.pallas import tpu as pltpu
from jax.experimental.pallas import tpu_sc as plsc

N = 10000
E = 320000
F = 128
NCLASS = 40

NC = 2
NS = 16
NW = NC * NS
CH = 128
NPAD = 10240
RPT = NPAD // NS
NCHUNKP = 2560
CPW = NCHUNKP // NW
CPT = NCHUNKP // NS
NB = 40

_MESH = plsc.VectorSubcoreMesh(core_axis_name="c", subcore_axis_name="s")


def _fill_vmem(ref, nrows, width, value):
    vv = jnp.full((16,), value, jnp.float32)
    def body(r, _):
        for j in range(width // 16):
            ref[r, pl.ds(j * 16, 16)] = vv
        return ()
    lax.fori_loop(0, nrows, body, ())


@functools.partial(
    pl.kernel,
    out_type=jax.ShapeDtypeStruct((2, NPAD), jnp.float32),
    mesh=_MESH,
    scratch_types=[
        pltpu.VMEM_SHARED((NPAD,), jnp.float32),
        pltpu.VMEM((NB, CH), jnp.int32),
        pltpu.VMEM((CH,), jnp.float32),
        pltpu.VMEM((RPT,), jnp.float32),
    ],
)
def _sc_degrees(ei2, out, acc, idxb, ones1, zbuf):
    cid = lax.axis_index("c")
    tid = lax.axis_index("s")

    zv = jnp.zeros((16,), jnp.float32)
    def z(r, _):
        zbuf[pl.ds(r * 16, 16)] = zv
        return ()
    lax.fori_loop(0, RPT // 16, z, ())
    pltpu.sync_copy(zbuf, acc.at[pl.ds(tid * RPT, RPT)])

    ov = jnp.ones((16,), jnp.float32)
    def o(r, _):
        ones1[pl.ds(r * 16, 16)] = ov
        return ()
    lax.fori_loop(0, CH // 16, o, ())

    plsc.subcore_barrier()

    base = cid * NCHUNKP + tid * CPT
    def batch(b, _):
        pltpu.sync_copy(ei2.at[pl.ds(base + b * NB, NB)], idxb)
        def step(j, _):
            pltpu.sync_copy(ones1, acc.at[idxb.at[j]], add=True)
            return ()
        lax.fori_loop(0, NB, step, ())
        return ()
    lax.fori_loop(0, CPT // NB, batch, ())

    plsc.subcore_barrier()
    pltpu.sync_copy(acc.at[pl.ds(tid * RPT, RPT)],
                    out.at[cid, pl.ds(tid * RPT, RPT)])


@functools.partial(
    pl.kernel,
    out_type=jax.ShapeDtypeStruct((NC, NPAD, F), jnp.float32),
    mesh=_MESH,
    scratch_types=[
        pltpu.VMEM_SHARED((NPAD, F), jnp.float32),
        pltpu.VMEM((NB, CH), jnp.int32),
        pltpu.VMEM((NB, CH), jnp.int32),
        pltpu.VMEM((CH, F), jnp.float32),
        pltpu.VMEM((CH, F), jnp.float32),
        pltpu.SemaphoreType.DMA,
        pltpu.SemaphoreType.DMA,
    ],
)
def _sc_aggregate(xs, ei2, out, acc, sidx, didx, rows0, rows1, sem0, sem1):
    cid = lax.axis_index("c")
    tid = lax.axis_index("s")
    wid = tid * NC + cid

    _fill_vmem(rows0, CH, F, 0.0)
    for k in range(RPT // CH):
        pltpu.sync_copy(rows0, acc.at[pl.ds(tid * RPT + k * CH, CH)])
    plsc.subcore_barrier()

    base = wid * CPW
    def half(h, _):
        hb = base + h * NB
        pltpu.sync_copy(ei2.at[pl.ds(hb, NB)], sidx)
        pltpu.sync_copy(ei2.at[pl.ds(NCHUNKP + hb, NB)], didx)
        pltpu.async_copy(xs.at[sidx.at[0]], rows0, sem0)
        def body(j2, _):
            c0 = 2 * j2
            pltpu.async_copy(xs.at[sidx.at[c0 + 1]], rows1, sem1)
            pltpu.make_async_copy(xs.at[sidx.at[c0]], rows0, sem0).wait()
            pltpu.sync_copy(rows0, acc.at[didx.at[c0]], add=True)
            @pl.when(j2 < NB // 2 - 1)
            def _():
                pltpu.async_copy(xs.at[sidx.at[c0 + 2]], rows0, sem0)
            pltpu.make_async_copy(xs.at[sidx.at[c0 + 1]], rows1, sem1).wait()
            pltpu.sync_copy(rows1, acc.at[didx.at[c0 + 1]], add=True)
            return ()
        lax.fori_loop(0, NB // 2, body, ())
        return ()
    lax.fori_loop(0, CPW // NB, half, ())

    plsc.subcore_barrier()
    pltpu.sync_copy(acc.at[pl.ds(tid * RPT, RPT)],
                    out.at[cid, pl.ds(tid * RPT, RPT)])


def _norm_from(deg_ref, which):
    d = deg_ref[which, :N]
    return jnp.where(d > 0.0, lax.rsqrt(jnp.maximum(d, 1.0)), 0.0)


def _tc_prescale_body(x_ref, deg_ref, o_ref):
    o_ref[:N] = x_ref[...] * _norm_from(deg_ref, 0)
    o_ref[N:] = jnp.zeros((NPAD - N, F), jnp.float32)


def _tc_layer1_body(p_ref, deg_ref, w_ref, b_ref, o_ref):
    y = (p_ref[0, :N] + p_ref[1, :N]) * _norm_from(deg_ref, 1)
    h = jnp.dot(y, w_ref[...], preferred_element_type=jnp.float32) + b_ref[...]
    h = jnp.maximum(h, 0.0)
    o_ref[:N] = h * _norm_from(deg_ref, 0)
    o_ref[N:] = jnp.zeros((NPAD - N, F), jnp.float32)


def _tc_layer2_body(p_ref, deg_ref, w_ref, b_ref, wfc_ref, bfc_ref, o_ref):
    y = (p_ref[0, :N] + p_ref[1, :N]) * _norm_from(deg_ref, 1)
    h = jnp.dot(y, w_ref[...], preferred_element_type=jnp.float32) + b_ref[...]
    o_ref[...] = (jnp.dot(h, wfc_ref[...], preferred_element_type=jnp.float32)
                  + bfc_ref[...])


def kernel(x, edge_index, W1, b1, W2, b2, Wfc, bfc):
    padv = N + jnp.arange(NCHUNKP * CH - E, dtype=jnp.int32) % (NPAD - N)
    ei2 = jnp.concatenate(
        [edge_index[0], padv, edge_index[1], padv]).reshape(2 * NCHUNKP, CH)

    degp = _sc_degrees(ei2)
    degc = degp.reshape(2, NPAD, 1)

    xs1 = pl.pallas_call(
        _tc_prescale_body,
        out_shape=jax.ShapeDtypeStruct((NPAD, F), jnp.float32),
    )(x, degc)

    P1 = _sc_aggregate(xs1, ei2)

    xs2 = pl.pallas_call(
        _tc_layer1_body,
        out_shape=jax.ShapeDtypeStruct((NPAD, F), jnp.float32),
    )(P1, degc, W1, b1.reshape(1, F))

    P2 = _sc_aggregate(xs2, ei2)

    out = pl.pallas_call(
        _tc_layer2_body,
        out_shape=jax.ShapeDtypeStruct((N, NCLASS), jnp.float32),
    )(P2, degc, W2, b2.reshape(1, F), Wfc, bfc.reshape(1, NCLASS))

    return out


# speedup vs baseline: 3.8818x; 1.0889x over previous
import functools

import jax
import jax.numpy as jnp
from jax import lax
from jax.experimental import pallas as pl
from jax.experimental.pallas import tpu as pltpu
from jax.experimental.pallas import tpu_sc as plsc

N = 10000
E = 320000
F = 128
NCLASS = 40

NC = 2
NS = 16
NW = NC * NS
CH = 128
NPAD = 10240
RPT = NPAD // NS
NCHUNKP = 2560
CPW = NCHUNKP // NW
CPT = NCHUNKP // NS
NB = 40

_MESH = plsc.VectorSubcoreMesh(core_axis_name="c", subcore_axis_name="s")


def _fill_vmem(ref, nrows, width, value):
    vv = jnp.full((16,), value, jnp.float32)
    def body(r, _):
        for j in range(width // 16):
            ref[r, pl.ds(j * 16, 16)] = vv
        return ()
    lax.fori_loop(0, nrows, body, ())


@functools.partial(
    pl.kernel,
    out_type=jax.ShapeDtypeStruct((2, NPAD), jnp.float32),
    mesh=_MESH,
    scratch_types=[
        pltpu.VMEM_SHARED((NPAD,), jnp.float32),
        pltpu.VMEM((NB, CH), jnp.int32),
        pltpu.VMEM((CH,), jnp.float32),
        pltpu.VMEM((RPT,), jnp.float32),
    ],
)
def _sc_degrees(ei2, out, acc, idxb, ones1, zbuf):
    cid = lax.axis_index("c")
    tid = lax.axis_index("s")

    zv = jnp.zeros((16,), jnp.float32)
    def z(r, _):
        zbuf[pl.ds(r * 16, 16)] = zv
        return ()
    lax.fori_loop(0, RPT // 16, z, ())
    pltpu.sync_copy(zbuf, acc.at[pl.ds(tid * RPT, RPT)])

    ov = jnp.ones((16,), jnp.float32)
    def o(r, _):
        ones1[pl.ds(r * 16, 16)] = ov
        return ()
    lax.fori_loop(0, CH // 16, o, ())

    plsc.subcore_barrier()

    base = cid * NCHUNKP + tid * CPT
    def batch(b, _):
        pltpu.sync_copy(ei2.at[pl.ds(base + b * NB, NB)], idxb)
        def step(j, _):
            pltpu.sync_copy(ones1, acc.at[idxb.at[j]], add=True)
            return ()
        lax.fori_loop(0, NB, step, ())
        return ()
    lax.fori_loop(0, CPT // NB, batch, ())

    plsc.subcore_barrier()
    pltpu.sync_copy(acc.at[pl.ds(tid * RPT, RPT)],
                    out.at[cid, pl.ds(tid * RPT, RPT)])


def _make_aggregate(width, compact):
    @functools.partial(
        pl.kernel,
        out_type=jax.ShapeDtypeStruct((NC, NPAD, width), jnp.float32),
        mesh=_MESH,
        compiler_params=(pltpu.CompilerParams(use_tc_tiling_on_sc=False)
                         if compact else None),
        scratch_types=[
            pltpu.VMEM_SHARED((NPAD, width), jnp.float32),
            pltpu.VMEM((NB, CH), jnp.int32),
            pltpu.VMEM((NB, CH), jnp.int32),
            pltpu.VMEM((CH, width), jnp.float32),
            pltpu.VMEM((CH, width), jnp.float32),
            pltpu.SemaphoreType.DMA,
            pltpu.SemaphoreType.DMA,
        ],
    )
    def agg(xs, ei2, out, acc, sidx, didx, rows0, rows1, sem0, sem1):
        cid = lax.axis_index("c")
        tid = lax.axis_index("s")
        wid = tid * NC + cid

        _fill_vmem(rows0, CH, width, 0.0)
        for k in range(RPT // CH):
            pltpu.sync_copy(rows0, acc.at[pl.ds(tid * RPT + k * CH, CH)])
        plsc.subcore_barrier()

        base = wid * CPW
        def half(h, _):
            hb = base + h * NB
            pltpu.sync_copy(ei2.at[pl.ds(hb, NB)], sidx)
            pltpu.sync_copy(ei2.at[pl.ds(NCHUNKP + hb, NB)], didx)
            pltpu.async_copy(xs.at[sidx.at[0]], rows0, sem0)
            def body(j2, _):
                c0 = 2 * j2
                pltpu.async_copy(xs.at[sidx.at[c0 + 1]], rows1, sem1)
                pltpu.make_async_copy(xs.at[sidx.at[c0]], rows0, sem0).wait()
                pltpu.sync_copy(rows0, acc.at[didx.at[c0]], add=True)
                @pl.when(j2 < NB // 2 - 1)
                def _():
                    pltpu.async_copy(xs.at[sidx.at[c0 + 2]], rows0, sem0)
                pltpu.make_async_copy(xs.at[sidx.at[c0 + 1]], rows1, sem1).wait()
                pltpu.sync_copy(rows1, acc.at[didx.at[c0 + 1]], add=True)
                return ()
            lax.fori_loop(0, NB // 2, body, ())
            return ()
        lax.fori_loop(0, CPW // NB, half, ())

        plsc.subcore_barrier()
        pltpu.sync_copy(acc.at[pl.ds(tid * RPT, RPT)],
                        out.at[cid, pl.ds(tid * RPT, RPT)])
    return agg


W2AGG = 64
_sc_aggregate = _make_aggregate(F, False)
_sc_aggregate_cls = _make_aggregate(W2AGG, True)


def _norm_from(deg_ref, which):
    d = deg_ref[which, :N]
    return jnp.where(d > 0.0, lax.rsqrt(jnp.maximum(d, 1.0)), 0.0)


def _tc_prescale_body(x_ref, deg_ref, o_ref):
    o_ref[:N] = x_ref[...] * _norm_from(deg_ref, 0)
    o_ref[N:] = jnp.zeros((NPAD - N, F), jnp.float32)


def _tc_layer1_body(p_ref, deg_ref, w1_ref, b1_ref, w2_ref, wfc_ref, o_ref):
    y = (p_ref[0, :N] + p_ref[1, :N]) * _norm_from(deg_ref, 1)
    h = jnp.dot(y, w1_ref[...], preferred_element_type=jnp.float32) + b1_ref[...]
    h = jnp.maximum(h, 0.0) * _norm_from(deg_ref, 0)
    w2f = jnp.dot(w2_ref[...], wfc_ref[...], preferred_element_type=jnp.float32)
    w2f = jnp.concatenate(
        [w2f, jnp.zeros((F, W2AGG - NCLASS), jnp.float32)], axis=1)
    o_ref[:N] = jnp.dot(h, w2f, preferred_element_type=jnp.float32)
    o_ref[N:] = jnp.zeros((NPAD - N, W2AGG), jnp.float32)


def _tc_final_body(p_ref, deg_ref, b2_ref, wfc_ref, bfc_ref, o_ref):
    y = (p_ref[0, :N, :NCLASS] + p_ref[1, :N, :NCLASS]) * _norm_from(deg_ref, 1)
    c = jnp.dot(b2_ref[...], wfc_ref[...], preferred_element_type=jnp.float32)
    o_ref[...] = y + c + bfc_ref[...]


def kernel(x, edge_index, W1, b1, W2, b2, Wfc, bfc):
    padv = N + jnp.arange(NCHUNKP * CH - E, dtype=jnp.int32) % (NPAD - N)
    ei2 = jnp.concatenate(
        [edge_index[0], padv, edge_index[1], padv]).reshape(2 * NCHUNKP, CH)

    degp = _sc_degrees(ei2)
    degc = degp.reshape(2, NPAD, 1)

    xs1 = pl.pallas_call(
        _tc_prescale_body,
        out_shape=jax.ShapeDtypeStruct((NPAD, F), jnp.float32),
    )(x, degc)

    P1 = _sc_aggregate(xs1, ei2)

    z2 = pl.pallas_call(
        _tc_layer1_body,
        out_shape=jax.ShapeDtypeStruct((NPAD, W2AGG), jnp.float32),
    )(P1, degc, W1, b1.reshape(1, F), W2, Wfc)

    P2 = _sc_aggregate_cls(z2, ei2)

    out = pl.pallas_call(
        _tc_final_body,
        out_shape=jax.ShapeDtypeStruct((N, NCLASS), jnp.float32),
    )(P2, degc, b2.reshape(1, F), Wfc, bfc.reshape(1, NCLASS))

    return out


# speedup vs baseline: 3.9725x; 1.0234x over previous
import functools

import jax
import jax.numpy as jnp
from jax import lax
from jax.experimental import pallas as pl
from jax.experimental.pallas import tpu as pltpu
from jax.experimental.pallas import tpu_sc as plsc

N = 10000
E = 320000
F = 128
NCLASS = 40

NC = 2
NS = 16
NW = NC * NS
CH = 128
NPAD = 10240
RPT = NPAD // NS
NCHUNKP = 2560
CPW = NCHUNKP // NW
CPT = NCHUNKP // NS
NB = 40

_MESH = plsc.VectorSubcoreMesh(core_axis_name="c", subcore_axis_name="s")


def _fill_vmem(ref, nrows, width, value):
    vv = jnp.full((16,), value, jnp.float32)
    def body(r, _):
        for j in range(width // 16):
            ref[r, pl.ds(j * 16, 16)] = vv
        return ()
    lax.fori_loop(0, nrows, body, ())


@functools.partial(
    pl.kernel,
    out_type=jax.ShapeDtypeStruct((2, NPAD), jnp.float32),
    mesh=_MESH,
    scratch_types=[
        pltpu.VMEM_SHARED((NPAD,), jnp.float32),
        pltpu.VMEM((CPT * CH,), jnp.int32),
        pltpu.VMEM((CPT * CH,), jnp.float32),
        pltpu.VMEM((RPT,), jnp.float32),
    ],
)
def _sc_degrees(eflat, out, acc, idxb, ones1, zbuf):
    cid = lax.axis_index("c")
    tid = lax.axis_index("s")

    zv = jnp.zeros((16,), jnp.float32)
    def z(r, _):
        zbuf[pl.ds(r * 16, 16)] = zv
        return ()
    lax.fori_loop(0, RPT // 16, z, ())
    pltpu.sync_copy(zbuf, acc.at[pl.ds(tid * RPT, RPT)])

    ov = jnp.ones((16,), jnp.float32)
    def o(r, _):
        ones1[pl.ds(r * 16, 16)] = ov
        return ()
    lax.fori_loop(0, CPT * CH // 16, o, ())

    plsc.subcore_barrier()

    base = (cid * NCHUNKP + tid * CPT) * CH
    pltpu.sync_copy(eflat.at[pl.ds(base, CPT * CH)], idxb)
    pltpu.sync_copy(ones1, acc.at[idxb], add=True)

    plsc.subcore_barrier()
    pltpu.sync_copy(acc.at[pl.ds(tid * RPT, RPT)],
                    out.at[cid, pl.ds(tid * RPT, RPT)])


def _make_aggregate(width, compact):
    @functools.partial(
        pl.kernel,
        out_type=jax.ShapeDtypeStruct((NC, NPAD, width), jnp.float32),
        mesh=_MESH,
        compiler_params=(pltpu.CompilerParams(use_tc_tiling_on_sc=False)
                         if compact else None),
        scratch_types=[
            pltpu.VMEM_SHARED((NPAD, width), jnp.float32),
            pltpu.VMEM((NB, CH), jnp.int32),
            pltpu.VMEM((NB, CH), jnp.int32),
            pltpu.VMEM((CH, width), jnp.float32),
            pltpu.VMEM((CH, width), jnp.float32),
            pltpu.SemaphoreType.DMA,
            pltpu.SemaphoreType.DMA,
        ],
    )
    def agg(xs, ei2, out, acc, sidx, didx, rows0, rows1, sem0, sem1):
        cid = lax.axis_index("c")
        tid = lax.axis_index("s")
        wid = tid * NC + cid

        _fill_vmem(rows0, CH, width, 0.0)
        for k in range(RPT // CH):
            pltpu.sync_copy(rows0, acc.at[pl.ds(tid * RPT + k * CH, CH)])
        plsc.subcore_barrier()

        base = wid * CPW
        def half(h, _):
            hb = base + h * NB
            pltpu.sync_copy(ei2.at[pl.ds(hb, NB)], sidx)
            pltpu.sync_copy(ei2.at[pl.ds(NCHUNKP + hb, NB)], didx)
            pltpu.async_copy(xs.at[sidx.at[0]], rows0, sem0)
            def body(j2, _):
                c0 = 2 * j2
                pltpu.async_copy(xs.at[sidx.at[c0 + 1]], rows1, sem1)
                pltpu.make_async_copy(xs.at[sidx.at[c0]], rows0, sem0).wait()
                pltpu.sync_copy(rows0, acc.at[didx.at[c0]], add=True)
                @pl.when(j2 < NB // 2 - 1)
                def _():
                    pltpu.async_copy(xs.at[sidx.at[c0 + 2]], rows0, sem0)
                pltpu.make_async_copy(xs.at[sidx.at[c0 + 1]], rows1, sem1).wait()
                pltpu.sync_copy(rows1, acc.at[didx.at[c0 + 1]], add=True)
                return ()
            lax.fori_loop(0, NB // 2, body, ())
            return ()
        lax.fori_loop(0, CPW // NB, half, ())

        plsc.subcore_barrier()
        pltpu.sync_copy(acc.at[pl.ds(tid * RPT, RPT)],
                        out.at[cid, pl.ds(tid * RPT, RPT)])
    return agg


W2AGG = 64
_sc_aggregate = _make_aggregate(F, False)
_sc_aggregate_cls = _make_aggregate(W2AGG, True)


def _norm_from(deg_ref, which):
    d = deg_ref[which, :N]
    return jnp.where(d > 0.0, lax.rsqrt(jnp.maximum(d, 1.0)), 0.0)


def _tc_prescale_body(x_ref, deg_ref, o_ref):
    o_ref[:N] = x_ref[...] * _norm_from(deg_ref, 0)
    o_ref[N:] = jnp.zeros((NPAD - N, F), jnp.float32)


def _tc_layer1_body(p_ref, deg_ref, w1_ref, b1_ref, w2_ref, wfc_ref, o_ref):
    y = (p_ref[0, :N] + p_ref[1, :N]) * _norm_from(deg_ref, 1)
    h = jnp.dot(y, w1_ref[...], preferred_element_type=jnp.float32) + b1_ref[...]
    h = jnp.maximum(h, 0.0) * _norm_from(deg_ref, 0)
    w2f = jnp.dot(w2_ref[...], wfc_ref[...], preferred_element_type=jnp.float32)
    w2f = jnp.concatenate(
        [w2f, jnp.zeros((F, W2AGG - NCLASS), jnp.float32)], axis=1)
    o_ref[:N] = jnp.dot(h, w2f, preferred_element_type=jnp.float32)
    o_ref[N:] = jnp.zeros((NPAD - N, W2AGG), jnp.float32)


def _tc_final_body(p_ref, deg_ref, b2_ref, wfc_ref, bfc_ref, o_ref):
    y = (p_ref[0, :N, :NCLASS] + p_ref[1, :N, :NCLASS]) * _norm_from(deg_ref, 1)
    c = jnp.dot(b2_ref[...], wfc_ref[...], preferred_element_type=jnp.float32)
    o_ref[...] = y + c + bfc_ref[...]


def kernel(x, edge_index, W1, b1, W2, b2, Wfc, bfc):
    padv = N + jnp.arange(NCHUNKP * CH - E, dtype=jnp.int32) % (NPAD - N)
    ei2 = jnp.concatenate(
        [edge_index[0], padv, edge_index[1], padv]).reshape(2 * NCHUNKP, CH)

    degp = _sc_degrees(ei2.reshape(-1))
    degc = degp.reshape(2, NPAD, 1)

    xs1 = pl.pallas_call(
        _tc_prescale_body,
        out_shape=jax.ShapeDtypeStruct((NPAD, F), jnp.float32),
    )(x, degc)

    P1 = _sc_aggregate(xs1, ei2)

    z2 = pl.pallas_call(
        _tc_layer1_body,
        out_shape=jax.ShapeDtypeStruct((NPAD, W2AGG), jnp.float32),
    )(P1, degc, W1, b1.reshape(1, F), W2, Wfc)

    P2 = _sc_aggregate_cls(z2, ei2)

    out = pl.pallas_call(
        _tc_final_body,
        out_shape=jax.ShapeDtypeStruct((N, NCLASS), jnp.float32),
    )(P2, degc, b2.reshape(1, F), Wfc, bfc.reshape(1, NCLASS))

    return out


# speedup vs baseline: 4.0008x; 1.0071x over previous
import functools

import jax
import jax.numpy as jnp
from jax import lax
from jax.experimental import pallas as pl
from jax.experimental.pallas import tpu as pltpu
from jax.experimental.pallas import tpu_sc as plsc

N = 10000
E = 320000
F = 128
NCLASS = 40

NC = 2
NS = 16
NW = NC * NS
CH = 128
NPAD = 10240
RPT = NPAD // NS
NCHUNKP = 2560
CPW = NCHUNKP // NW
CPT = NCHUNKP // NS
NB = 40

_MESH = plsc.VectorSubcoreMesh(core_axis_name="c", subcore_axis_name="s")


def _fill_vmem(ref, nrows, width, value):
    vv = jnp.full((16,), value, jnp.float32)
    def body(r, _):
        for j in range(width // 16):
            ref[r, pl.ds(j * 16, 16)] = vv
        return ()
    lax.fori_loop(0, nrows, body, ())


@functools.partial(
    pl.kernel,
    out_type=jax.ShapeDtypeStruct((2, NPAD), jnp.float32),
    mesh=_MESH,
    scratch_types=[
        pltpu.VMEM_SHARED((NPAD,), jnp.float32),
        pltpu.VMEM((CPT * CH,), jnp.int32),
        pltpu.VMEM((CPT * CH,), jnp.float32),
        pltpu.VMEM((RPT,), jnp.float32),
    ],
)
def _sc_degrees(eflat, out, acc, idxb, ones1, zbuf):
    cid = lax.axis_index("c")
    tid = lax.axis_index("s")

    zv = jnp.zeros((16,), jnp.float32)
    def z(r, _):
        zbuf[pl.ds(r * 16, 16)] = zv
        return ()
    lax.fori_loop(0, RPT // 16, z, ())
    pltpu.sync_copy(zbuf, acc.at[pl.ds(tid * RPT, RPT)])

    ov = jnp.ones((16,), jnp.float32)
    def o(r, _):
        ones1[pl.ds(r * 16, 16)] = ov
        return ()
    lax.fori_loop(0, CPT * CH // 16, o, ())

    plsc.subcore_barrier()

    base = (cid * NCHUNKP + tid * CPT) * CH
    pltpu.sync_copy(eflat.at[pl.ds(base, CPT * CH)], idxb)
    pltpu.sync_copy(ones1, acc.at[idxb], add=True)

    plsc.subcore_barrier()
    pltpu.sync_copy(acc.at[pl.ds(tid * RPT, RPT)],
                    out.at[cid, pl.ds(tid * RPT, RPT)])


def _make_aggregate(width, compact, nb):
    @functools.partial(
        pl.kernel,
        out_type=jax.ShapeDtypeStruct((NC, NPAD, width), jnp.float32),
        mesh=_MESH,
        compiler_params=(pltpu.CompilerParams(use_tc_tiling_on_sc=False)
                         if compact else None),
        scratch_types=[
            pltpu.VMEM_SHARED((NPAD, width), jnp.float32),
            pltpu.VMEM((nb, CH), jnp.int32),
            pltpu.VMEM((nb, CH), jnp.int32),
            pltpu.VMEM((CH, width), jnp.float32),
            pltpu.VMEM((CH, width), jnp.float32),
            pltpu.SemaphoreType.DMA,
            pltpu.SemaphoreType.DMA,
        ],
    )
    def agg(xs, ei2, out, acc, sidx, didx, rows0, rows1, sem0, sem1):
        cid = lax.axis_index("c")
        tid = lax.axis_index("s")
        wid = tid * NC + cid

        _fill_vmem(rows0, CH, width, 0.0)
        for k in range(RPT // CH):
            pltpu.sync_copy(rows0, acc.at[pl.ds(tid * RPT + k * CH, CH)])
        plsc.subcore_barrier()

        base = wid * CPW
        def half(h, _):
            hb = base + h * nb
            pltpu.sync_copy(ei2.at[pl.ds(hb, nb)], sidx)
            pltpu.sync_copy(ei2.at[pl.ds(NCHUNKP + hb, nb)], didx)
            pltpu.async_copy(xs.at[sidx.at[0]], rows0, sem0)
            def body(j2, _):
                c0 = 2 * j2
                pltpu.async_copy(xs.at[sidx.at[c0 + 1]], rows1, sem1)
                pltpu.make_async_copy(xs.at[sidx.at[c0]], rows0, sem0).wait()
                pltpu.sync_copy(rows0, acc.at[didx.at[c0]], add=True)
                @pl.when(j2 < nb // 2 - 1)
                def _():
                    pltpu.async_copy(xs.at[sidx.at[c0 + 2]], rows0, sem0)
                pltpu.make_async_copy(xs.at[sidx.at[c0 + 1]], rows1, sem1).wait()
                pltpu.sync_copy(rows1, acc.at[didx.at[c0 + 1]], add=True)
                return ()
            lax.fori_loop(0, nb // 2, body, ())
            return ()
        lax.fori_loop(0, CPW // nb, half, ())

        plsc.subcore_barrier()
        pltpu.sync_copy(acc.at[pl.ds(tid * RPT, RPT)],
                        out.at[cid, pl.ds(tid * RPT, RPT)])
    return agg


W2AGG = 64
_sc_aggregate = _make_aggregate(F, False, NB)
_sc_aggregate_cls = _make_aggregate(W2AGG, True, CPW)


def _norm_from(deg_ref, which):
    d = deg_ref[which, :N]
    return jnp.where(d > 0.0, lax.rsqrt(jnp.maximum(d, 1.0)), 0.0)


def _tc_prescale_body(x_ref, deg_ref, o_ref):
    o_ref[:N] = x_ref[...] * _norm_from(deg_ref, 0)
    o_ref[N:] = jnp.zeros((NPAD - N, F), jnp.float32)


def _tc_layer1_body(p_ref, deg_ref, w1_ref, b1_ref, w2_ref, wfc_ref, o_ref):
    y = (p_ref[0, :N] + p_ref[1, :N]) * _norm_from(deg_ref, 1)
    h = jnp.dot(y, w1_ref[...], preferred_element_type=jnp.float32) + b1_ref[...]
    h = jnp.maximum(h, 0.0) * _norm_from(deg_ref, 0)
    w2f = jnp.dot(w2_ref[...], wfc_ref[...], preferred_element_type=jnp.float32)
    w2f = jnp.concatenate(
        [w2f, jnp.zeros((F, W2AGG - NCLASS), jnp.float32)], axis=1)
    o_ref[:N] = jnp.dot(h, w2f, preferred_element_type=jnp.float32)
    o_ref[N:] = jnp.zeros((NPAD - N, W2AGG), jnp.float32)


def _tc_final_body(p_ref, deg_ref, b2_ref, wfc_ref, bfc_ref, o_ref):
    y = (p_ref[0, :N, :NCLASS] + p_ref[1, :N, :NCLASS]) * _norm_from(deg_ref, 1)
    c = jnp.dot(b2_ref[...], wfc_ref[...], preferred_element_type=jnp.float32)
    o_ref[...] = y + c + bfc_ref[...]


def kernel(x, edge_index, W1, b1, W2, b2, Wfc, bfc):
    padv = N + jnp.arange(NCHUNKP * CH - E, dtype=jnp.int32) % (NPAD - N)
    ei2 = jnp.concatenate(
        [edge_index[0], padv, edge_index[1], padv]).reshape(2 * NCHUNKP, CH)

    degp = _sc_degrees(ei2.reshape(-1))
    degc = degp.reshape(2, NPAD, 1)

    xs1 = pl.pallas_call(
        _tc_prescale_body,
        out_shape=jax.ShapeDtypeStruct((NPAD, F), jnp.float32),
    )(x, degc)

    P1 = _sc_aggregate(xs1, ei2)

    z2 = pl.pallas_call(
        _tc_layer1_body,
        out_shape=jax.ShapeDtypeStruct((NPAD, W2AGG), jnp.float32),
    )(P1, degc, W1, b1.reshape(1, F), W2, Wfc)

    P2 = _sc_aggregate_cls(z2, ei2)

    out = pl.pallas_call(
        _tc_final_body,
        out_shape=jax.ShapeDtypeStruct((N, NCLASS), jnp.float32),
    )(P2, degc, b2.reshape(1, F), Wfc, bfc.reshape(1, NCLASS))

    return out


# speedup vs baseline: 4.0601x; 1.0148x over previous
import functools

import jax
import jax.numpy as jnp
from jax import lax
from jax.experimental import pallas as pl
from jax.experimental.pallas import tpu as pltpu
from jax.experimental.pallas import tpu_sc as plsc

N = 10000
E = 320000
F = 128
NCLASS = 40

NC = 2
NS = 16
NW = NC * NS
CH = 128
NPAD = 10240
RPT = NPAD // NS
NCHUNKP = 2560
CPW = NCHUNKP // NW
CPT = NCHUNKP // NS
NB = 40

_MESH = plsc.VectorSubcoreMesh(core_axis_name="c", subcore_axis_name="s")


def _fill_vmem(ref, nrows, width, value):
    vv = jnp.full((16,), value, jnp.float32)
    def body(r, _):
        for j in range(width // 16):
            ref[r, pl.ds(j * 16, 16)] = vv
        return ()
    lax.fori_loop(0, nrows, body, ())


@functools.partial(
    pl.kernel,
    out_type=jax.ShapeDtypeStruct((2, NPAD), jnp.float32),
    mesh=_MESH,
    scratch_types=[
        pltpu.VMEM_SHARED((NPAD,), jnp.float32),
        pltpu.VMEM((CPT * CH,), jnp.int32),
        pltpu.VMEM((CPT * CH,), jnp.float32),
        pltpu.VMEM((RPT,), jnp.float32),
    ],
)
def _sc_degrees(eflat, out, acc, idxb, ones1, zbuf):
    cid = lax.axis_index("c")
    tid = lax.axis_index("s")

    zv = jnp.zeros((16,), jnp.float32)
    def z(r, _):
        for j in range(8):
            zbuf[pl.ds(r * 128 + j * 16, 16)] = zv
        return ()
    lax.fori_loop(0, RPT // 128, z, ())
    pltpu.sync_copy(zbuf, acc.at[pl.ds(tid * RPT, RPT)])

    ov = jnp.ones((16,), jnp.float32)
    def o(r, _):
        for j in range(8):
            ones1[pl.ds(r * 128 + j * 16, 16)] = ov
        return ()
    lax.fori_loop(0, CPT * CH // 128, o, ())

    plsc.subcore_barrier()

    base = (cid * NCHUNKP + tid * CPT) * CH
    pltpu.sync_copy(eflat.at[pl.ds(base, CPT * CH)], idxb)
    pltpu.sync_copy(ones1, acc.at[idxb], add=True)

    plsc.subcore_barrier()
    pltpu.sync_copy(acc.at[pl.ds(tid * RPT, RPT)],
                    out.at[cid, pl.ds(tid * RPT, RPT)])


def _make_aggregate(width, compact, nb):
    @functools.partial(
        pl.kernel,
        out_type=jax.ShapeDtypeStruct((NC, NPAD, width), jnp.float32),
        mesh=_MESH,
        compiler_params=(pltpu.CompilerParams(use_tc_tiling_on_sc=False)
                         if compact else None),
        scratch_types=[
            pltpu.VMEM_SHARED((NPAD, width), jnp.float32),
            pltpu.VMEM((nb, CH), jnp.int32),
            pltpu.VMEM((nb, CH), jnp.int32),
            pltpu.VMEM((CH, width), jnp.float32),
            pltpu.VMEM((CH, width), jnp.float32),
            pltpu.SemaphoreType.DMA,
            pltpu.SemaphoreType.DMA,
        ],
    )
    def agg(xs, ei2, out, acc, sidx, didx, rows0, rows1, sem0, sem1):
        cid = lax.axis_index("c")
        tid = lax.axis_index("s")
        wid = tid * NC + cid

        _fill_vmem(rows0, CH, width, 0.0)
        for k in range(RPT // CH):
            pltpu.sync_copy(rows0, acc.at[pl.ds(tid * RPT + k * CH, CH)])
        plsc.subcore_barrier()

        base = wid * CPW
        def half(h, _):
            hb = base + h * nb
            pltpu.sync_copy(ei2.at[pl.ds(hb, nb)], sidx)
            pltpu.sync_copy(ei2.at[pl.ds(NCHUNKP + hb, nb)], didx)
            pltpu.async_copy(xs.at[sidx.at[0]], rows0, sem0)
            def body(j2, _):
                c0 = 2 * j2
                pltpu.async_copy(xs.at[sidx.at[c0 + 1]], rows1, sem1)
                pltpu.make_async_copy(xs.at[sidx.at[c0]], rows0, sem0).wait()
                pltpu.sync_copy(rows0, acc.at[didx.at[c0]], add=True)
                @pl.when(j2 < nb // 2 - 1)
                def _():
                    pltpu.async_copy(xs.at[sidx.at[c0 + 2]], rows0, sem0)
                pltpu.make_async_copy(xs.at[sidx.at[c0 + 1]], rows1, sem1).wait()
                pltpu.sync_copy(rows1, acc.at[didx.at[c0 + 1]], add=True)
                return ()
            lax.fori_loop(0, nb // 2, body, ())
            return ()
        lax.fori_loop(0, CPW // nb, half, ())

        plsc.subcore_barrier()
        pltpu.sync_copy(acc.at[pl.ds(tid * RPT, RPT)],
                        out.at[cid, pl.ds(tid * RPT, RPT)])
    return agg


W2AGG = 64
_sc_aggregate = _make_aggregate(F, False, NB)
_sc_aggregate_cls = _make_aggregate(W2AGG, True, CPW)


def _norm_from(deg_ref, which):
    d = deg_ref[which, :N]
    return jnp.where(d > 0.0, lax.rsqrt(jnp.maximum(d, 1.0)), 0.0)


def _tc_prescale_body(x_ref, deg_ref, o_ref):
    o_ref[:N] = x_ref[...] * _norm_from(deg_ref, 0)
    o_ref[N:] = jnp.zeros((NPAD - N, F), jnp.float32)


def _tc_layer1_body(p_ref, deg_ref, w1_ref, b1_ref, w2_ref, wfc_ref, o_ref):
    y = (p_ref[0, :N] + p_ref[1, :N]) * _norm_from(deg_ref, 1)
    h = jnp.dot(y, w1_ref[...], preferred_element_type=jnp.float32) + b1_ref[...]
    h = jnp.maximum(h, 0.0) * _norm_from(deg_ref, 0)
    w2f = jnp.dot(w2_ref[...], wfc_ref[...], preferred_element_type=jnp.float32)
    w2f = jnp.concatenate(
        [w2f, jnp.zeros((F, W2AGG - NCLASS), jnp.float32)], axis=1)
    o_ref[:N] = jnp.dot(h, w2f, preferred_element_type=jnp.float32)
    o_ref[N:] = jnp.zeros((NPAD - N, W2AGG), jnp.float32)


def _tc_final_body(p_ref, deg_ref, b2_ref, wfc_ref, bfc_ref, o_ref):
    y = (p_ref[0, :N, :NCLASS] + p_ref[1, :N, :NCLASS]) * _norm_from(deg_ref, 1)
    c = jnp.dot(b2_ref[...], wfc_ref[...], preferred_element_type=jnp.float32)
    o_ref[...] = y + c + bfc_ref[...]


def kernel(x, edge_index, W1, b1, W2, b2, Wfc, bfc):
    padv = N + jnp.arange(NCHUNKP * CH - E, dtype=jnp.int32) % (NPAD - N)
    ei2 = jnp.concatenate(
        [edge_index[0], padv, edge_index[1], padv]).reshape(2 * NCHUNKP, CH)

    degp = _sc_degrees(ei2.reshape(-1))
    degc = degp.reshape(2, NPAD, 1)

    xs1 = pl.pallas_call(
        _tc_prescale_body,
        out_shape=jax.ShapeDtypeStruct((NPAD, F), jnp.float32),
    )(x, degc)

    P1 = _sc_aggregate(xs1, ei2)

    z2 = pl.pallas_call(
        _tc_layer1_body,
        out_shape=jax.ShapeDtypeStruct((NPAD, W2AGG), jnp.float32),
    )(P1, degc, W1, b1.reshape(1, F), W2, Wfc)

    P2 = _sc_aggregate_cls(z2, ei2)

    out = pl.pallas_call(
        _tc_final_body,
        out_shape=jax.ShapeDtypeStruct((N, NCLASS), jnp.float32),
    )(P2, degc, b2.reshape(1, F), Wfc, bfc.reshape(1, NCLASS))

    return out


# speedup vs baseline: 4.1530x; 1.0229x over previous
import functools

import jax
import jax.numpy as jnp
from jax import lax
from jax.experimental import pallas as pl
from jax.experimental.pallas import tpu as pltpu
from jax.experimental.pallas import tpu_sc as plsc

N = 10000
E = 320000
F = 128
NCLASS = 40

NC = 2
NS = 16
NW = NC * NS
CH = 128
NPAD = 10240
RPT = NPAD // NS
NCHUNKP = 2560
CPW = NCHUNKP // NW
CPT = NCHUNKP // NS
NB = 40
EPT = E // NS

_MESH = plsc.VectorSubcoreMesh(core_axis_name="c", subcore_axis_name="s")


def _fill_vmem(ref, nrows, width, value):
    vv = jnp.full((16,), value, jnp.float32)
    def body(r, _):
        for j in range(width // 16):
            ref[r, pl.ds(j * 16, 16)] = vv
        return ()
    lax.fori_loop(0, nrows, body, ())


@functools.partial(
    pl.kernel,
    out_type=jax.ShapeDtypeStruct((2, NPAD), jnp.float32),
    mesh=_MESH,
    scratch_types=[
        pltpu.VMEM_SHARED((NPAD,), jnp.float32),
        pltpu.VMEM((EPT,), jnp.int32),
        pltpu.VMEM((EPT,), jnp.float32),
        pltpu.VMEM((RPT,), jnp.float32),
    ],
)
def _sc_degrees(eflat, out, acc, idxb, ones1, zbuf):
    cid = lax.axis_index("c")
    tid = lax.axis_index("s")

    zv = jnp.zeros((16,), jnp.float32)
    def z(r, _):
        for j in range(8):
            zbuf[pl.ds(r * 128 + j * 16, 16)] = zv
        return ()
    lax.fori_loop(0, RPT // 128, z, ())
    pltpu.sync_copy(zbuf, acc.at[pl.ds(tid * RPT, RPT)])

    ov = jnp.ones((16,), jnp.float32)
    def o(r, _):
        for j in range(8):
            ones1[pl.ds(r * 128 + j * 16, 16)] = ov
        return ()
    lax.fori_loop(0, EPT // 128, o, ())

    plsc.subcore_barrier()

    base = cid * E + tid * EPT
    pltpu.sync_copy(eflat.at[pl.ds(base, EPT)], idxb)
    pltpu.sync_copy(ones1, acc.at[idxb], add=True)

    plsc.subcore_barrier()
    pltpu.sync_copy(acc.at[pl.ds(tid * RPT, RPT)],
                    out.at[cid, pl.ds(tid * RPT, RPT)])


def _make_aggregate(width, compact, nb):
    @functools.partial(
        pl.kernel,
        out_type=jax.ShapeDtypeStruct((NC, NPAD, width), jnp.float32),
        mesh=_MESH,
        compiler_params=(pltpu.CompilerParams(use_tc_tiling_on_sc=False)
                         if compact else None),
        scratch_types=[
            pltpu.VMEM_SHARED((NPAD, width), jnp.float32),
            pltpu.VMEM((nb, CH), jnp.int32),
            pltpu.VMEM((nb, CH), jnp.int32),
            pltpu.VMEM((CH, width), jnp.float32),
            pltpu.VMEM((CH, width), jnp.float32),
            pltpu.SemaphoreType.DMA,
            pltpu.SemaphoreType.DMA,
        ],
    )
    def agg(xs, ei2, out, acc, sidx, didx, rows0, rows1, sem0, sem1):
        cid = lax.axis_index("c")
        tid = lax.axis_index("s")
        wid = tid * NC + cid

        _fill_vmem(rows0, CH, width, 0.0)
        for k in range(RPT // CH):
            pltpu.sync_copy(rows0, acc.at[pl.ds(tid * RPT + k * CH, CH)])
        plsc.subcore_barrier()

        base = wid * CPW
        def half(h, _):
            hb = base + h * nb
            pltpu.sync_copy(ei2.at[pl.ds(hb, nb)], sidx)
            pltpu.sync_copy(ei2.at[pl.ds(NCHUNKP + hb, nb)], didx)
            pltpu.async_copy(xs.at[sidx.at[0]], rows0, sem0)
            def body(j2, _):
                c0 = 2 * j2
                pltpu.async_copy(xs.at[sidx.at[c0 + 1]], rows1, sem1)
                pltpu.make_async_copy(xs.at[sidx.at[c0]], rows0, sem0).wait()
                pltpu.sync_copy(rows0, acc.at[didx.at[c0]], add=True)
                @pl.when(j2 < nb // 2 - 1)
                def _():
                    pltpu.async_copy(xs.at[sidx.at[c0 + 2]], rows0, sem0)
                pltpu.make_async_copy(xs.at[sidx.at[c0 + 1]], rows1, sem1).wait()
                pltpu.sync_copy(rows1, acc.at[didx.at[c0 + 1]], add=True)
                return ()
            lax.fori_loop(0, nb // 2, body, ())
            return ()
        lax.fori_loop(0, CPW // nb, half, ())

        plsc.subcore_barrier()
        pltpu.sync_copy(acc.at[pl.ds(tid * RPT, RPT)],
                        out.at[cid, pl.ds(tid * RPT, RPT)])
    return agg


W2AGG = 64
_sc_aggregate = _make_aggregate(F, False, NB)
_sc_aggregate_cls = _make_aggregate(W2AGG, True, CPW)


def _norm_from(deg_ref, which):
    d = deg_ref[which, :N]
    return jnp.where(d > 0.0, lax.rsqrt(jnp.maximum(d, 1.0)), 0.0)


def _tc_prescale_body(x_ref, deg_ref, o_ref):
    o_ref[:N] = x_ref[...] * _norm_from(deg_ref, 0)
    o_ref[N:] = jnp.zeros((NPAD - N, F), jnp.float32)


def _tc_layer1_body(p_ref, deg_ref, w1_ref, b1_ref, w2_ref, wfc_ref, o_ref):
    y = (p_ref[0, :N] + p_ref[1, :N]) * _norm_from(deg_ref, 1)
    h = jnp.dot(y, w1_ref[...], preferred_element_type=jnp.float32) + b1_ref[...]
    h = jnp.maximum(h, 0.0) * _norm_from(deg_ref, 0)
    w2f = jnp.dot(w2_ref[...], wfc_ref[...], preferred_element_type=jnp.float32)
    w2f = jnp.concatenate(
        [w2f, jnp.zeros((F, W2AGG - NCLASS), jnp.float32)], axis=1)
    o_ref[:N] = jnp.dot(h, w2f, preferred_element_type=jnp.float32)
    o_ref[N:] = jnp.zeros((NPAD - N, W2AGG), jnp.float32)


def _tc_final_body(p_ref, deg_ref, b2_ref, wfc_ref, bfc_ref, o_ref):
    y = (p_ref[0, :N, :NCLASS] + p_ref[1, :N, :NCLASS]) * _norm_from(deg_ref, 1)
    c = jnp.dot(b2_ref[...], wfc_ref[...], preferred_element_type=jnp.float32)
    o_ref[...] = y + c + bfc_ref[...]


def kernel(x, edge_index, W1, b1, W2, b2, Wfc, bfc):
    padv = N + jnp.arange(NCHUNKP * CH - E, dtype=jnp.int32) % (NPAD - N)
    ei2 = jnp.concatenate(
        [edge_index[0], padv, edge_index[1], padv]).reshape(2 * NCHUNKP, CH)

    degp = _sc_degrees(edge_index.reshape(-1))
    degc = degp.reshape(2, NPAD, 1)

    xs1 = pl.pallas_call(
        _tc_prescale_body,
        out_shape=jax.ShapeDtypeStruct((NPAD, F), jnp.float32),
    )(x, degc)

    P1 = _sc_aggregate(xs1, ei2)

    z2 = pl.pallas_call(
        _tc_layer1_body,
        out_shape=jax.ShapeDtypeStruct((NPAD, W2AGG), jnp.float32),
    )(P1, degc, W1, b1.reshape(1, F), W2, Wfc)

    P2 = _sc_aggregate_cls(z2, ei2)

    out = pl.pallas_call(
        _tc_final_body,
        out_shape=jax.ShapeDtypeStruct((N, NCLASS), jnp.float32),
    )(P2, degc, b2.reshape(1, F), Wfc, bfc.reshape(1, NCLASS))

    return out


# speedup vs baseline: 4.1533x; 1.0001x over previous
import functools

import jax
import jax.numpy as jnp
from jax import lax
from jax.experimental import pallas as pl
from jax.experimental.pallas import tpu as pltpu
from jax.experimental.pallas import tpu_sc as plsc

N = 10000
E = 320000
F = 128
NCLASS = 40

NC = 2
NS = 16
NW = NC * NS
CH = 128
NPAD = 10240
RPT = NPAD // NS
NCHUNKP = 2560
CPW = NCHUNKP // NW
CPT = NCHUNKP // NS
NB = 40
EPT = E // NS

_MESH = plsc.VectorSubcoreMesh(core_axis_name="c", subcore_axis_name="s")


def _fill_vmem(ref, nrows, width, value):
    vv = jnp.full((16,), value, jnp.float32)
    def body(r, _):
        for j in range(width // 16):
            ref[r, pl.ds(j * 16, 16)] = vv
        return ()
    lax.fori_loop(0, nrows, body, ())


@functools.partial(
    pl.kernel,
    out_type=jax.ShapeDtypeStruct((2, NPAD), jnp.float32),
    mesh=_MESH,
    scratch_types=[
        pltpu.VMEM_SHARED((NPAD,), jnp.float32),
        pltpu.VMEM((EPT,), jnp.int32),
        pltpu.VMEM((EPT,), jnp.float32),
        pltpu.VMEM((RPT,), jnp.float32),
    ],
)
def _sc_degrees(eflat, out, acc, idxb, ones1, zbuf):
    cid = lax.axis_index("c")
    tid = lax.axis_index("s")

    zv = jnp.zeros((16,), jnp.float32)
    def z(r, _):
        for j in range(8):
            zbuf[pl.ds(r * 128 + j * 16, 16)] = zv
        return ()
    lax.fori_loop(0, RPT // 128, z, ())
    pltpu.sync_copy(zbuf, acc.at[pl.ds(tid * RPT, RPT)])

    ov = jnp.ones((16,), jnp.float32)
    def o(r, _):
        for j in range(10):
            ones1[pl.ds(r * 160 + j * 16, 16)] = ov
        return ()
    lax.fori_loop(0, EPT // 160, o, ())

    plsc.subcore_barrier()

    base = cid * E + tid * EPT
    pltpu.sync_copy(eflat.at[pl.ds(base, EPT)], idxb)
    pltpu.sync_copy(ones1, acc.at[idxb], add=True)

    plsc.subcore_barrier()
    pltpu.sync_copy(acc.at[pl.ds(tid * RPT, RPT)],
                    out.at[cid, pl.ds(tid * RPT, RPT)])


def _make_aggregate(width, compact, nb):
    @functools.partial(
        pl.kernel,
        out_type=jax.ShapeDtypeStruct((NC, NPAD, width), jnp.float32),
        mesh=_MESH,
        compiler_params=(pltpu.CompilerParams(use_tc_tiling_on_sc=False)
                         if compact else None),
        scratch_types=[
            pltpu.VMEM_SHARED((NPAD, width), jnp.float32),
            pltpu.VMEM((nb, CH), jnp.int32),
            pltpu.VMEM((nb, CH), jnp.int32),
            pltpu.VMEM((CH, width), jnp.float32),
            pltpu.VMEM((CH, width), jnp.float32),
            pltpu.SemaphoreType.DMA,
            pltpu.SemaphoreType.DMA,
        ],
    )
    def agg(xs, ei2, out, acc, sidx, didx, rows0, rows1, sem0, sem1):
        cid = lax.axis_index("c")
        tid = lax.axis_index("s")
        wid = tid * NC + cid

        _fill_vmem(rows0, CH, width, 0.0)
        for k in range(RPT // CH):
            pltpu.sync_copy(rows0, acc.at[pl.ds(tid * RPT + k * CH, CH)])
        plsc.subcore_barrier()

        base = wid * CPW
        def half(h, _):
            hb = base + h * nb
            pltpu.sync_copy(ei2.at[pl.ds(hb, nb)], sidx)
            pltpu.sync_copy(ei2.at[pl.ds(NCHUNKP + hb, nb)], didx)
            pltpu.async_copy(xs.at[sidx.at[0]], rows0, sem0)
            def body(j2, _):
                c0 = 2 * j2
                pltpu.async_copy(xs.at[sidx.at[c0 + 1]], rows1, sem1)
                pltpu.make_async_copy(xs.at[sidx.at[c0]], rows0, sem0).wait()
                pltpu.sync_copy(rows0, acc.at[didx.at[c0]], add=True)
                @pl.when(j2 < nb // 2 - 1)
                def _():
                    pltpu.async_copy(xs.at[sidx.at[c0 + 2]], rows0, sem0)
                pltpu.make_async_copy(xs.at[sidx.at[c0 + 1]], rows1, sem1).wait()
                pltpu.sync_copy(rows1, acc.at[didx.at[c0 + 1]], add=True)
                return ()
            lax.fori_loop(0, nb // 2, body, ())
            return ()
        lax.fori_loop(0, CPW // nb, half, ())

        plsc.subcore_barrier()
        pltpu.sync_copy(acc.at[pl.ds(tid * RPT, RPT)],
                        out.at[cid, pl.ds(tid * RPT, RPT)])
    return agg


W2AGG = 64
_sc_aggregate = _make_aggregate(F, False, NB)
_sc_aggregate_cls = _make_aggregate(W2AGG, True, CPW)


def _norm_from(deg_ref, which):
    d = deg_ref[which, :N]
    return jnp.where(d > 0.0, lax.rsqrt(jnp.maximum(d, 1.0)), 0.0)


def _tc_prescale_body(x_ref, deg_ref, o_ref):
    o_ref[:N] = x_ref[...] * _norm_from(deg_ref, 0)
    o_ref[N:] = jnp.zeros((NPAD - N, F), jnp.float32)


def _tc_layer1_body(p_ref, deg_ref, w1_ref, b1_ref, w2_ref, wfc_ref, o_ref):
    y = (p_ref[0, :N] + p_ref[1, :N]) * _norm_from(deg_ref, 1)
    h = jnp.dot(y, w1_ref[...], preferred_element_type=jnp.float32) + b1_ref[...]
    h = jnp.maximum(h, 0.0) * _norm_from(deg_ref, 0)
    w2f = jnp.dot(w2_ref[...], wfc_ref[...], preferred_element_type=jnp.float32)
    w2f = jnp.concatenate(
        [w2f, jnp.zeros((F, W2AGG - NCLASS), jnp.float32)], axis=1)
    o_ref[:N] = jnp.dot(h, w2f, preferred_element_type=jnp.float32)
    o_ref[N:] = jnp.zeros((NPAD - N, W2AGG), jnp.float32)


def _tc_final_body(p_ref, deg_ref, b2_ref, wfc_ref, bfc_ref, o_ref):
    y = (p_ref[0, :N, :NCLASS] + p_ref[1, :N, :NCLASS]) * _norm_from(deg_ref, 1)
    c = jnp.dot(b2_ref[...], wfc_ref[...], preferred_element_type=jnp.float32)
    o_ref[...] = y + c + bfc_ref[...]


def kernel(x, edge_index, W1, b1, W2, b2, Wfc, bfc):
    padv = N + jnp.arange(NCHUNKP * CH - E, dtype=jnp.int32) % (NPAD - N)
    ei2 = jnp.concatenate(
        [edge_index[0], padv, edge_index[1], padv]).reshape(2 * NCHUNKP, CH)

    degp = _sc_degrees(edge_index.reshape(-1))
    degc = degp.reshape(2, NPAD, 1)

    xs1 = pl.pallas_call(
        _tc_prescale_body,
        out_shape=jax.ShapeDtypeStruct((NPAD, F), jnp.float32),
    )(x, degc)

    P1 = _sc_aggregate(xs1, ei2)

    z2 = pl.pallas_call(
        _tc_layer1_body,
        out_shape=jax.ShapeDtypeStruct((NPAD, W2AGG), jnp.float32),
    )(P1, degc, W1, b1.reshape(1, F), W2, Wfc)

    P2 = _sc_aggregate_cls(z2, ei2)

    out = pl.pallas_call(
        _tc_final_body,
        out_shape=jax.ShapeDtypeStruct((N, NCLASS), jnp.float32),
    )(P2, degc, b2.reshape(1, F), Wfc, bfc.reshape(1, NCLASS))

    return out
